# baseline (speedup 1.0000x reference)
import jax
import jax.numpy as jnp
from jax.experimental import pallas as pl

_NC = 1000


def _onehot_block(x_ref, o_ref):
    iota = jax.lax.broadcasted_iota(jnp.int32, o_ref.shape, 1)
    o_ref[...] = (x_ref[...] == iota).astype(jnp.float32)


def kernel(x):
    B, S = x.shape
    N = B * S
    R = 1024
    y = pl.pallas_call(
        _onehot_block,
        grid=(N // R,),
        in_specs=[pl.BlockSpec((R, 1), lambda i: (i, 0))],
        out_specs=pl.BlockSpec((R, _NC), lambda i: (i, 0)),
        out_shape=jax.ShapeDtypeStruct((N, _NC), jnp.float32),
    )(x.reshape(N, 1))
    return y.reshape(B, S, _NC)


# speedup vs baseline: 1.4665x; 1.4665x over previous
import jax
import jax.numpy as jnp
from jax.experimental import pallas as pl
from jax.experimental.pallas import tpu as pltpu

_NC = 1000
_R = 16
_NBUF = 12


def _onehot_pipelined(x_ref, o_ref, buf, sem):
    i = pl.program_id(0)
    nblocks = pl.num_programs(0)
    slot = jax.lax.rem(i, _NBUF)
    s, nc = x_ref.shape[1], _NC

    @pl.when(i >= _NBUF)
    def _():
        pltpu.make_async_copy(
            buf.at[slot], o_ref.at[pl.ds(0, _R)], sem.at[slot]
        ).wait()

    xv = x_ref[pl.ds(i * _R, _R), :]
    iota = jax.lax.broadcasted_iota(jnp.int32, (_R, s, nc), 2)
    buf[slot] = (xv[:, :, None] == iota).astype(jnp.float32)
    pltpu.make_async_copy(
        buf.at[slot], o_ref.at[pl.ds(i * _R, _R)], sem.at[slot]
    ).start()

    @pl.when(i == nblocks - 1)
    def _():
        for j in range(_NBUF):
            pltpu.make_async_copy(
                buf.at[j], o_ref.at[pl.ds(0, _R)], sem.at[j]
            ).wait()


def kernel(x):
    B, S = x.shape
    return pl.pallas_call(
        _onehot_pipelined,
        grid=(B // _R,),
        in_specs=[pl.BlockSpec((B, S), lambda i: (0, 0))],
        out_specs=pl.BlockSpec(memory_space=pl.ANY),
        out_shape=jax.ShapeDtypeStruct((B, S, _NC), jnp.float32),
        scratch_shapes=[
            pltpu.VMEM((_NBUF, _R, S, _NC), jnp.float32),
            pltpu.SemaphoreType.DMA((_NBUF,)),
        ],
    )(x)
